# baseline (speedup 1.0000x reference)
import jax
import jax.numpy as jnp
from jax.experimental import pallas as pl
from jax.experimental.pallas import tpu as pltpu

_VMEM = 64 * 1024 * 1024


def _cp(*sem):
    return pltpu.CompilerParams(dimension_semantics=tuple(sem),
                                vmem_limit_bytes=_VMEM)


def _make_conv_body(k, nin, act, with_mean, th, wo, cout):
    def body(*refs):
        x_refs = refs[:nin]
        w_refs = refs[nin:2 * nin]
        b_ref = refs[2 * nin]
        m_ref = refs[2 * nin + 1] if with_mean else None
        o_ref = refs[-1]
        r0 = pl.program_id(1) * th
        acc = jnp.zeros((th * wo, cout), jnp.float32)
        for x_ref, w_ref in zip(x_refs, w_refs):
            cin = x_ref.shape[-1]
            for i in range(k):
                rows = x_ref[0, pl.ds(r0 + i, th), :, :]
                for j in range(k):
                    a = rows[:, j:j + wo, :].reshape(th * wo, cin)
                    acc = acc + jnp.dot(a, w_ref[i, j],
                                        preferred_element_type=jnp.float32)
        acc = acc + b_ref[...]
        if with_mean:
            acc = acc + m_ref[0]
        if act == "lrelu":
            acc = jnp.where(acc >= 0.0, acc, 0.1 * acc)
        elif act == "clamp":
            acc = jnp.clip(acc, 0.0, 1.0)
        o_ref[0] = acc.reshape(th, wo, cout).astype(o_ref.dtype)
    return body


def _conv(xs, offs, ws, bias, *, k, nb, act=None, out_dtype=jnp.bfloat16,
          mean=None):
    Hp, Wp = xs[0].shape[1], xs[0].shape[2]
    Ho, Wo = Hp - k + 1, Wp - k + 1
    cout = ws[0].shape[-1]
    th = min(Ho, max(8, 2048 // Wo))
    body = _make_conv_body(k, len(xs), act, mean is not None, th, Wo, cout)
    in_specs, operands = [], []
    for x, off in zip(xs, offs):
        in_specs.append(pl.BlockSpec((1, Hp, Wp, x.shape[-1]),
                                     (lambda b, r, _o=off: (b + _o, 0, 0, 0))))
        operands.append(x)
    for w in ws:
        in_specs.append(pl.BlockSpec(w.shape, lambda b, r: (0, 0, 0, 0)))
        operands.append(w)
    in_specs.append(pl.BlockSpec((1, cout), lambda b, r: (0, 0)))
    operands.append(bias)
    if mean is not None:
        in_specs.append(pl.BlockSpec((1, 1, cout), lambda b, r: (b, 0, 0)))
        operands.append(mean)
    return pl.pallas_call(
        body,
        out_shape=jax.ShapeDtypeStruct((nb, Ho, Wo, cout), out_dtype),
        grid=(nb, Ho // th),
        in_specs=in_specs,
        out_specs=pl.BlockSpec((1, th, Wo, cout), lambda b, r: (b, r, 0, 0)),
        compiler_params=_cp("parallel", "arbitrary"),
    )(*operands)


_PARITIES = ((0, 0), (0, 1), (1, 0), (1, 1))


def _make_convt_body(th, wo, cout):
    def body(x_ref, w0, w1, w2, w3, b_ref, o0, o1, o2, o3):
        r0 = pl.program_id(1) * th
        w_refs = (w0, w1, w2, w3)
        o_refs = (o0, o1, o2, o3)
        cin = x_ref.shape[-1]
        bias = b_ref[...]
        for p, (py, px) in enumerate(_PARITIES):
            acc = jnp.zeros((th * wo, cout), jnp.float32)
            for a in range(2):
                rows = x_ref[0, pl.ds(r0 + a + py, th), :, :]
                for b2 in range(2):
                    c0 = b2 + px
                    aa = rows[:, c0:c0 + wo, :].reshape(th * wo, cin)
                    acc = acc + jnp.dot(aa, w_refs[p][a, b2],
                                        preferred_element_type=jnp.float32)
            acc = acc + bias
            o_refs[p][0] = acc.reshape(th, wo, cout).astype(o_refs[p].dtype)
    return body


def _conv_transpose(xp, ws4, bias):
    B, Hp, Wp, cin = xp.shape
    Ho, Wo = Hp - 2, Wp - 2
    cout = ws4[0].shape[-1]
    th = min(Ho, max(8, 2048 // Wo))
    oshape = jax.ShapeDtypeStruct((B, Ho, Wo, cout), jnp.bfloat16)
    ys = pl.pallas_call(
        _make_convt_body(th, Wo, cout),
        out_shape=(oshape,) * 4,
        grid=(B, Ho // th),
        in_specs=[pl.BlockSpec((1, Hp, Wp, cin), lambda b, r: (b, 0, 0, 0))]
        + [pl.BlockSpec((2, 2, cin, cout), lambda b, r: (0, 0, 0, 0))] * 4
        + [pl.BlockSpec((1, cout), lambda b, r: (0, 0))],
        out_specs=tuple(pl.BlockSpec((1, th, Wo, cout),
                                     lambda b, r: (b, r, 0, 0))
                        for _ in range(4)),
        compiler_params=_cp("parallel", "arbitrary"),
    )(xp, *ws4, bias)
    y = jnp.stack(ys, axis=3).reshape(B, Ho, Wo, 2, 2, cout)
    y = jnp.transpose(y, (0, 1, 3, 2, 4, 5)).reshape(B, 2 * Ho, 2 * Wo, cout)
    return y


def _make_head_body(th, ho, wo, chr_, cout):
    hwid = wo + 2

    def body(x_ref, whr_ref, bhr_ref, wcl_ref, bcl_ref, m_ref, o_ref):
        r0 = pl.program_id(1) * th
        acch = jnp.zeros(((th + 2) * hwid, chr_), jnp.float32)
        for i in range(3):
            rows = x_ref[0, pl.ds(r0 + i, th + 2), :, :]
            for j in range(3):
                a = rows[:, j:j + hwid, :].reshape((th + 2) * hwid, chr_)
                acch = acch + jnp.dot(a, whr_ref[i, j],
                                      preferred_element_type=jnp.float32)
        acch = acch + bhr_ref[...]
        acch = jnp.where(acch >= 0.0, acch, 0.1 * acch)
        hr = acch.reshape(th + 2, hwid, chr_)
        rid = jax.lax.broadcasted_iota(jnp.int32, hr.shape, 0) + (r0 - 1)
        cid = jax.lax.broadcasted_iota(jnp.int32, hr.shape, 1) - 1
        valid = ((rid >= 0) & (rid < ho) & (cid >= 0) & (cid < wo))
        hr = jnp.where(valid, hr, 0.0).astype(jnp.bfloat16)
        acc = jnp.zeros((th * wo, cout), jnp.float32)
        for i in range(3):
            for j in range(3):
                a = hr[i:i + th, j:j + wo, :].reshape(th * wo, chr_)
                acc = acc + jnp.dot(a, wcl_ref[i, j],
                                    preferred_element_type=jnp.float32)
        acc = acc + bcl_ref[...] + m_ref[0]
        o_ref[0] = jnp.clip(acc, 0.0, 1.0).reshape(th, wo, cout)
    return body


def _head_tail(ups, whr, bhr, wcl, bcl, mean_b):
    B, Ho, Wo, Chr = ups.shape
    cout = wcl.shape[-1]
    xp2 = jnp.pad(ups, ((0, 0), (2, 2), (2, 2), (0, 0)))
    th = 8
    return pl.pallas_call(
        _make_head_body(th, Ho, Wo, Chr, cout),
        out_shape=jax.ShapeDtypeStruct((B, Ho, Wo, cout), jnp.float32),
        grid=(B, Ho // th),
        in_specs=[
            pl.BlockSpec((1, Ho + 4, Wo + 4, Chr), lambda b, r: (b, 0, 0, 0)),
            pl.BlockSpec(whr.shape, lambda b, r: (0, 0, 0, 0)),
            pl.BlockSpec((1, Chr), lambda b, r: (0, 0)),
            pl.BlockSpec(wcl.shape, lambda b, r: (0, 0, 0, 0)),
            pl.BlockSpec((1, cout), lambda b, r: (0, 0)),
            pl.BlockSpec((1, 1, cout), lambda b, r: (b, 0, 0)),
        ],
        out_specs=pl.BlockSpec((1, th, Wo, cout), lambda b, r: (b, r, 0, 0)),
        compiler_params=_cp("parallel", "arbitrary"),
    )(xp2, whr, bhr, wcl, bcl, mean_b)


def _make_norm_body(hw):
    def body(x0_ref, x1_ref, y_ref, m_ref):
        a = x0_ref[...]
        b = x1_ref[...]
        s = jnp.sum(a, axis=1, keepdims=True) + jnp.sum(b, axis=1, keepdims=True)
        m = s * (1.0 / (2.0 * hw * 255.0))
        y_ref[0] = (a * (1.0 / 255.0) - m).astype(y_ref.dtype)
        y_ref[1] = (b * (1.0 / 255.0) - m).astype(y_ref.dtype)
        m_ref[...] = m
    return body


def _normalize(x0, x1):
    B, C, H, W = x0.shape
    R, HW = B * C, H * W
    TR = 8 if R % 8 == 0 else R
    y, mean = pl.pallas_call(
        _make_norm_body(HW),
        out_shape=(jax.ShapeDtypeStruct((2, R, HW), jnp.bfloat16),
                   jax.ShapeDtypeStruct((R, 1), jnp.float32)),
        grid=(R // TR,),
        in_specs=[pl.BlockSpec((TR, HW), lambda i: (i, 0)),
                  pl.BlockSpec((TR, HW), lambda i: (i, 0))],
        out_specs=(pl.BlockSpec((2, TR, HW), lambda i: (0, i, 0)),
                   pl.BlockSpec((TR, 1), lambda i: (i, 0))),
        compiler_params=_cp("parallel"),
    )(x0.reshape(R, HW), x1.reshape(R, HW))
    return y.reshape(2 * B, C, H, W), mean


def _pad1(x):
    return jnp.pad(x, ((0, 0), (1, 1), (1, 1), (0, 0)))


def _s2d(xp):
    N, Hp, Wp, C = xp.shape
    x = xp.reshape(N, Hp // 2, 2, Wp // 2, 2, C)
    x = jnp.transpose(x, (0, 1, 3, 2, 4, 5))
    return x.reshape(N, Hp // 2, Wp // 2, 4 * C)


def _w_s2d(wmat, cin, cout):
    w9 = wmat.reshape(3, 3, cin, cout)
    w2 = jnp.zeros((2, 2, 4 * cin, cout), wmat.dtype)
    for i in range(3):
        for j in range(3):
            c0 = (i % 2) * 2 * cin + (j % 2) * cin
            w2 = w2.at[i // 2, j // 2, c0:c0 + cin, :].set(w9[i, j])
    return w2


def _pixel_shuffle(x, r=2):
    B, H, W, C = x.shape
    c = C // (r * r)
    x = x.reshape(B, H, W, c, r, r)
    x = jnp.transpose(x, (0, 1, 4, 2, 5, 3))
    return x.reshape(B, H * r, W * r, c)


def _make_warp_body(hw, w, h, c, qc):
    nq = hw // qc

    def body(img0_ref, img1_ref, flo_ref, o_ref):
        for wi, img_ref in enumerate((img0_ref, img1_ref)):
            sx = flo_ref[0, wi, 0]
            sy = flo_ref[0, wi, 1]
            x0f = jnp.floor(sx)
            y0f = jnp.floor(sy)
            wx = sx - x0f
            wy = sy - y0f
            acc = jnp.zeros((hw, c), jnp.float32)
            for dy, dx in ((0, 0), (0, 1), (1, 0), (1, 1)):
                yc = jnp.clip(y0f + dy, 0, h - 1).astype(jnp.int32)
                xc = jnp.clip(x0f + dx, 0, w - 1).astype(jnp.int32)
                idx = yc * w + xc
                wgt = ((wx if dx else 1.0 - wx)
                       * (wy if dy else 1.0 - wy))
                sampled = jnp.zeros((hw, c), jnp.float32)
                for q0 in range(nq):
                    q = q0 * qc + jax.lax.broadcasted_iota(
                        jnp.int32, (hw, qc), 1)
                    s = (idx == q).astype(jnp.bfloat16)
                    sampled = sampled + jnp.dot(
                        s, img_ref[0, q0 * qc:(q0 + 1) * qc, :],
                        preferred_element_type=jnp.float32)
                acc = acc + wgt * sampled
            o_ref[wi, 0] = acc.astype(o_ref.dtype)
    return body


def _bwarp2(f3, ft0, ft1):
    twoB, Hs, Ws, C = f3.shape
    B = twoB // 2
    HW = Hs * Ws
    gy, gx = jnp.meshgrid(jnp.arange(Hs, dtype=jnp.float32),
                          jnp.arange(Ws, dtype=jnp.float32), indexing="ij")
    g = jnp.stack([gx, gy])[None]
    flo = jnp.stack([g + ft0, g + ft1], axis=1)
    flo = flo.reshape(B, 2, 2, HW, 1)
    imgs = f3.reshape(twoB, HW, C)
    out = pl.pallas_call(
        _make_warp_body(HW, Ws, Hs, C, min(256, HW)),
        out_shape=jax.ShapeDtypeStruct((2, B, HW, C), jnp.bfloat16),
        grid=(B,),
        in_specs=[pl.BlockSpec((1, HW, C), lambda b: (b, 0, 0)),
                  pl.BlockSpec((1, HW, C), lambda b: (b + B, 0, 0)),
                  pl.BlockSpec((1, 2, 2, HW, 1), lambda b: (b, 0, 0, 0, 0))],
        out_specs=pl.BlockSpec((2, 1, HW, C), lambda b: (0, b, 0, 0)),
        compiler_params=_cp("parallel"),
    )(imgs, imgs, flo)
    return (out[0].reshape(B, Hs, Ws, C), out[1].reshape(B, Hs, Ws, C))


def _nhwc(x):
    return jnp.transpose(x, (0, 2, 3, 1))


def _nchw(x):
    return jnp.transpose(x, (0, 3, 1, 2))


def kernel(enc1_w, enc1_b, enc2_w, enc2_b, enc3_w, enc3_b, enc4_w, enc4_b,
           dec4_w, dec4_b, qb3_w, qb3_b, dec3_w, dec3_b, dec2_w, dec2_b,
           dec1_w, dec1_b, upconv1_w, upconv1_b, hrconv_w, hrconv_b,
           convlast_w, convlast_b,
           qb2_ws0, qb2_ws1, qb2_ws2, qb2_ws3, qb2_b,
           qb1_ws0, qb1_ws1, qb1_ws2, qb1_ws3, qb1_b,
           x0, x1, t):
    B, C, H, W = x0.shape

    xcat, mean = _normalize(x0, x1)

    xin = _s2d(_pad1(_nhwc(xcat)))
    f1 = _conv([xin], [0], [_w_s2d(enc1_w, C, 32)], enc1_b, k=2, nb=2 * B,
               act="lrelu")
    f1p = _pad1(f1)
    f2 = _conv([_s2d(f1p)], [0], [_w_s2d(enc2_w, 32, 48)], enc2_b, k=2,
               nb=2 * B, act="lrelu")
    f2p = _pad1(f2)
    f3 = _conv([_s2d(f2p)], [0], [_w_s2d(enc3_w, 48, 72)], enc3_b, k=2,
               nb=2 * B, act="lrelu")
    f3p = _pad1(f3)
    f4 = _conv([f3p], [0], [enc4_w.reshape(3, 3, 72, 96)], enc4_b, k=3,
               nb=2 * B, act="lrelu")

    f4p = _pad1(f4)
    wd4 = dec4_w.reshape(3, 3, 192, 4)
    out4 = _conv([f4p, f4p], [0, B], [wd4[:, :, :96], wd4[:, :, 96:]], dec4_b,
                 k=3, nb=B, out_dtype=jnp.float32)
    f01 = _nchw(out4[..., 0:2])
    t4 = t.reshape(B, 1, 1, 1)
    ft0 = -(f01 * t4) * t4
    ft1 = -(f01 * (1.0 - t4)) * (1.0 - t4)

    warp0, warp1 = _bwarp2(f3, ft0, ft1)
    w0p = _pad1(warp0)
    w1p = _pad1(warp1)
    wq3 = qb3_w.reshape(3, 3, 144, 72)
    q3 = _conv([w0p, w1p], [0, 0], [wq3[:, :, :72], wq3[:, :, 72:]], qb3_b,
               k=3, nb=B)
    wd3 = dec3_w.reshape(3, 3, 216, 72)
    p3 = _conv([_pad1(q3), f3p, f3p], [0, 0, B],
               [wd3[:, :, :72], wd3[:, :, 72:144], wd3[:, :, 144:]], dec3_b,
               k=3, nb=B, act="lrelu")

    q2 = _conv_transpose(_pad1(p3),
                         [w.reshape(2, 2, 72, 48)
                          for w in (qb2_ws0, qb2_ws1, qb2_ws2, qb2_ws3)],
                         qb2_b)
    wd2 = dec2_w.reshape(3, 3, 144, 48)
    p2 = _conv([_pad1(q2), f2p, f2p], [0, 0, B],
               [wd2[:, :, :48], wd2[:, :, 48:96], wd2[:, :, 96:]], dec2_b,
               k=3, nb=B, act="lrelu")

    q1 = _conv_transpose(_pad1(p2),
                         [w.reshape(2, 2, 48, 32)
                          for w in (qb1_ws0, qb1_ws1, qb1_ws2, qb1_ws3)],
                         qb1_b)
    wd1 = dec1_w.reshape(3, 3, 96, 32)
    p1 = _conv([_pad1(q1), f1p, f1p], [0, 0, B],
               [wd1[:, :, :32], wd1[:, :, 32:64], wd1[:, :, 64:]], dec1_b,
               k=3, nb=B, act="lrelu")

    up = _conv([_pad1(p1)], [0], [upconv1_w.reshape(3, 3, 32, 128)],
               upconv1_b, k=3, nb=B, act="lrelu")
    ups = _pixel_shuffle(up, 2)
    mean_b = mean.reshape(B, C)[:, None, :]
    out = _head_tail(ups, hrconv_w.reshape(3, 3, 32, 32), hrconv_b,
                     convlast_w.reshape(3, 3, 32, 3), convlast_b, mean_b)
    return _nchw(out)


# speedup vs baseline: 1.6603x; 1.6603x over previous
import jax
import jax.numpy as jnp
from jax.experimental import pallas as pl
from jax.experimental.pallas import tpu as pltpu

_VMEM = 64 * 1024 * 1024


def _cp(*sem):
    return pltpu.CompilerParams(dimension_semantics=tuple(sem),
                                vmem_limit_bytes=_VMEM)


def _make_conv_body(k, nin, act, with_mean, th, wo, cout):
    def body(*refs):
        x_refs = refs[:nin]
        w_refs = refs[nin:2 * nin]
        b_ref = refs[2 * nin]
        m_ref = refs[2 * nin + 1] if with_mean else None
        o_ref = refs[-1]
        r0 = pl.program_id(1) * th
        acc = jnp.zeros((th * wo, cout), jnp.float32)
        for x_ref, w_ref in zip(x_refs, w_refs):
            cin = x_ref.shape[-1]
            for i in range(k):
                rows = x_ref[0, pl.ds(r0 + i, th), :, :]
                for j in range(k):
                    a = rows[:, j:j + wo, :].reshape(th * wo, cin)
                    acc = acc + jnp.dot(a, w_ref[i, j],
                                        preferred_element_type=jnp.float32)
        acc = acc + b_ref[...]
        if with_mean:
            acc = acc + m_ref[0]
        if act == "lrelu":
            acc = jnp.where(acc >= 0.0, acc, 0.1 * acc)
        elif act == "clamp":
            acc = jnp.clip(acc, 0.0, 1.0)
        o_ref[0] = acc.reshape(th, wo, cout).astype(o_ref.dtype)
    return body


def _conv(xs, offs, ws, bias, *, k, nb, act=None, out_dtype=jnp.bfloat16,
          mean=None):
    Hp, Wp = xs[0].shape[1], xs[0].shape[2]
    Ho, Wo = Hp - k + 1, Wp - k + 1
    cout = ws[0].shape[-1]
    th = min(Ho, max(8, 2048 // Wo))
    body = _make_conv_body(k, len(xs), act, mean is not None, th, Wo, cout)
    in_specs, operands = [], []
    for x, off in zip(xs, offs):
        in_specs.append(pl.BlockSpec((1, Hp, Wp, x.shape[-1]),
                                     (lambda b, r, _o=off: (b + _o, 0, 0, 0))))
        operands.append(x)
    for w in ws:
        in_specs.append(pl.BlockSpec(w.shape, lambda b, r: (0, 0, 0, 0)))
        operands.append(w)
    in_specs.append(pl.BlockSpec((1, cout), lambda b, r: (0, 0)))
    operands.append(bias)
    if mean is not None:
        in_specs.append(pl.BlockSpec((1, 1, cout), lambda b, r: (b, 0, 0)))
        operands.append(mean)
    return pl.pallas_call(
        body,
        out_shape=jax.ShapeDtypeStruct((nb, Ho, Wo, cout), out_dtype),
        grid=(nb, Ho // th),
        in_specs=in_specs,
        out_specs=pl.BlockSpec((1, th, Wo, cout), lambda b, r: (b, r, 0, 0)),
        compiler_params=_cp("parallel", "arbitrary"),
    )(*operands)


_PARITIES = ((0, 0), (0, 1), (1, 0), (1, 1))


def _make_convt_body(th, wo, cout):
    def body(x_ref, w0, w1, w2, w3, b_ref, o0, o1, o2, o3):
        r0 = pl.program_id(1) * th
        w_refs = (w0, w1, w2, w3)
        o_refs = (o0, o1, o2, o3)
        cin = x_ref.shape[-1]
        bias = b_ref[...]
        for p, (py, px) in enumerate(_PARITIES):
            acc = jnp.zeros((th * wo, cout), jnp.float32)
            for a in range(2):
                rows = x_ref[0, pl.ds(r0 + a + py, th), :, :]
                for b2 in range(2):
                    c0 = b2 + px
                    aa = rows[:, c0:c0 + wo, :].reshape(th * wo, cin)
                    acc = acc + jnp.dot(aa, w_refs[p][a, b2],
                                        preferred_element_type=jnp.float32)
            acc = acc + bias
            o_refs[p][0] = acc.reshape(th, wo, cout).astype(o_refs[p].dtype)
    return body


def _conv_transpose(xp, ws4, bias):
    B, Hp, Wp, cin = xp.shape
    Ho, Wo = Hp - 2, Wp - 2
    cout = ws4[0].shape[-1]
    th = min(Ho, max(8, 2048 // Wo))
    oshape = jax.ShapeDtypeStruct((B, Ho, Wo, cout), jnp.bfloat16)
    ys = pl.pallas_call(
        _make_convt_body(th, Wo, cout),
        out_shape=(oshape,) * 4,
        grid=(B, Ho // th),
        in_specs=[pl.BlockSpec((1, Hp, Wp, cin), lambda b, r: (b, 0, 0, 0))]
        + [pl.BlockSpec((2, 2, cin, cout), lambda b, r: (0, 0, 0, 0))] * 4
        + [pl.BlockSpec((1, cout), lambda b, r: (0, 0))],
        out_specs=tuple(pl.BlockSpec((1, th, Wo, cout),
                                     lambda b, r: (b, r, 0, 0))
                        for _ in range(4)),
        compiler_params=_cp("parallel", "arbitrary"),
    )(xp, *ws4, bias)
    y = jnp.stack(ys, axis=3).reshape(B, Ho, Wo, 2, 2, cout)
    y = jnp.transpose(y, (0, 1, 3, 2, 4, 5)).reshape(B, 2 * Ho, 2 * Wo, cout)
    return y


def _make_head_body(th, ho, wo, chr_, cout):
    hwid = wo + 2

    def body(x_ref, whr_ref, bhr_ref, wcl_ref, bcl_ref, m_ref, o_ref):
        r0 = pl.program_id(1) * th
        acch = jnp.zeros(((th + 2) * hwid, chr_), jnp.float32)
        for i in range(3):
            rows = x_ref[0, pl.ds(r0 + i, th + 2), :, :]
            for j in range(3):
                a = rows[:, j:j + hwid, :].reshape((th + 2) * hwid, chr_)
                acch = acch + jnp.dot(a, whr_ref[i, j],
                                      preferred_element_type=jnp.float32)
        acch = acch + bhr_ref[...]
        acch = jnp.where(acch >= 0.0, acch, 0.1 * acch)
        hr = acch.reshape(th + 2, hwid, chr_)
        rid = jax.lax.broadcasted_iota(jnp.int32, hr.shape, 0) + (r0 - 1)
        cid = jax.lax.broadcasted_iota(jnp.int32, hr.shape, 1) - 1
        valid = ((rid >= 0) & (rid < ho) & (cid >= 0) & (cid < wo))
        hr = jnp.where(valid, hr, 0.0).astype(jnp.bfloat16)
        acc = jnp.zeros((th * wo, cout), jnp.float32)
        for i in range(3):
            for j in range(3):
                a = hr[i:i + th, j:j + wo, :].reshape(th * wo, chr_)
                acc = acc + jnp.dot(a, wcl_ref[i, j],
                                    preferred_element_type=jnp.float32)
        acc = acc + bcl_ref[...] + m_ref[0]
        o_ref[0] = jnp.clip(acc, 0.0, 1.0).reshape(th, wo, cout)
    return body


def _head_tail(ups, whr, bhr, wcl, bcl, mean_b):
    B, Ho, Wo, Chr = ups.shape
    cout = wcl.shape[-1]
    xp2 = jnp.pad(ups, ((0, 0), (2, 2), (2, 2), (0, 0)))
    th = 8
    return pl.pallas_call(
        _make_head_body(th, Ho, Wo, Chr, cout),
        out_shape=jax.ShapeDtypeStruct((B, Ho, Wo, cout), jnp.float32),
        grid=(B, Ho // th),
        in_specs=[
            pl.BlockSpec((1, Ho + 4, Wo + 4, Chr), lambda b, r: (b, 0, 0, 0)),
            pl.BlockSpec(whr.shape, lambda b, r: (0, 0, 0, 0)),
            pl.BlockSpec((1, Chr), lambda b, r: (0, 0)),
            pl.BlockSpec(wcl.shape, lambda b, r: (0, 0, 0, 0)),
            pl.BlockSpec((1, cout), lambda b, r: (0, 0)),
            pl.BlockSpec((1, 1, cout), lambda b, r: (b, 0, 0)),
        ],
        out_specs=pl.BlockSpec((1, th, Wo, cout), lambda b, r: (b, r, 0, 0)),
        compiler_params=_cp("parallel", "arbitrary"),
    )(xp2, whr, bhr, wcl, bcl, mean_b)


def _make_norm_body(hw):
    def body(x0_ref, x1_ref, y_ref, m_ref):
        a = x0_ref[...]
        b = x1_ref[...]
        s = jnp.sum(a, axis=1, keepdims=True) + jnp.sum(b, axis=1, keepdims=True)
        m = s * (1.0 / (2.0 * hw * 255.0))
        y_ref[0] = (a * (1.0 / 255.0) - m).astype(y_ref.dtype)
        y_ref[1] = (b * (1.0 / 255.0) - m).astype(y_ref.dtype)
        m_ref[...] = m
    return body


def _normalize(x0, x1):
    B, C, H, W = x0.shape
    R, HW = B * C, H * W
    TR = 8 if R % 8 == 0 else R
    y, mean = pl.pallas_call(
        _make_norm_body(HW),
        out_shape=(jax.ShapeDtypeStruct((2, R, HW), jnp.bfloat16),
                   jax.ShapeDtypeStruct((R, 1), jnp.float32)),
        grid=(R // TR,),
        in_specs=[pl.BlockSpec((TR, HW), lambda i: (i, 0)),
                  pl.BlockSpec((TR, HW), lambda i: (i, 0))],
        out_specs=(pl.BlockSpec((2, TR, HW), lambda i: (0, i, 0)),
                   pl.BlockSpec((TR, 1), lambda i: (i, 0))),
        compiler_params=_cp("parallel"),
    )(x0.reshape(R, HW), x1.reshape(R, HW))
    return y.reshape(2 * B, C, H, W), mean


def _pad1(x):
    return jnp.pad(x, ((0, 0), (1, 1), (1, 1), (0, 0)))


def _s2d(xp):
    N, Hp, Wp, C = xp.shape
    x = xp.reshape(N, Hp // 2, 2, Wp // 2, 2, C)
    x = jnp.transpose(x, (0, 1, 3, 2, 4, 5))
    return x.reshape(N, Hp // 2, Wp // 2, 4 * C)


def _w_s2d(wmat, cin, cout):
    w9 = wmat.reshape(3, 3, cin, cout)
    w2 = jnp.zeros((2, 2, 4 * cin, cout), wmat.dtype)
    for i in range(3):
        for j in range(3):
            c0 = (i % 2) * 2 * cin + (j % 2) * cin
            w2 = w2.at[i // 2, j // 2, c0:c0 + cin, :].set(w9[i, j])
    return w2


def _pixel_shuffle(x, r=2):
    B, H, W, C = x.shape
    c = C // (r * r)
    x = x.reshape(B, H, W, c, r, r)
    x = jnp.transpose(x, (0, 1, 4, 2, 5, 3))
    return x.reshape(B, H * r, W * r, c)


def _make_warp_body(hw, w, h, c, qc):
    nq = hw // qc

    def body(img0_ref, img1_ref, flo_ref, o_ref):
        for wi, img_ref in enumerate((img0_ref, img1_ref)):
            sx = flo_ref[0, wi, 0]
            sy = flo_ref[0, wi, 1]
            x0f = jnp.floor(sx)
            y0f = jnp.floor(sy)
            wx = sx - x0f
            wy = sy - y0f
            acc = jnp.zeros((hw, c), jnp.float32)
            for dy, dx in ((0, 0), (0, 1), (1, 0), (1, 1)):
                yc = jnp.clip(y0f + dy, 0, h - 1).astype(jnp.int32)
                xc = jnp.clip(x0f + dx, 0, w - 1).astype(jnp.int32)
                idx = yc * w + xc
                wgt = ((wx if dx else 1.0 - wx)
                       * (wy if dy else 1.0 - wy))
                sampled = jnp.zeros((hw, c), jnp.float32)
                for q0 in range(nq):
                    q = q0 * qc + jax.lax.broadcasted_iota(
                        jnp.int32, (hw, qc), 1)
                    s = (idx == q).astype(jnp.bfloat16)
                    sampled = sampled + jnp.dot(
                        s, img_ref[0, q0 * qc:(q0 + 1) * qc, :],
                        preferred_element_type=jnp.float32)
                acc = acc + wgt * sampled
            o_ref[wi, 0] = acc.astype(o_ref.dtype)
    return body


def _bwarp2(f3, ft0, ft1):
    twoB, Hs, Ws, C = f3.shape
    B = twoB // 2
    HW = Hs * Ws
    gy, gx = jnp.meshgrid(jnp.arange(Hs, dtype=jnp.float32),
                          jnp.arange(Ws, dtype=jnp.float32), indexing="ij")
    g = jnp.stack([gx, gy])[None]
    flo = jnp.stack([g + ft0, g + ft1], axis=1)
    flo = flo.reshape(B, 2, 2, HW, 1)
    imgs = f3.reshape(twoB, HW, C)
    out = pl.pallas_call(
        _make_warp_body(HW, Ws, Hs, C, min(256, HW)),
        out_shape=jax.ShapeDtypeStruct((2, B, HW, C), jnp.bfloat16),
        grid=(B,),
        in_specs=[pl.BlockSpec((1, HW, C), lambda b: (b, 0, 0)),
                  pl.BlockSpec((1, HW, C), lambda b: (b + B, 0, 0)),
                  pl.BlockSpec((1, 2, 2, HW, 1), lambda b: (b, 0, 0, 0, 0))],
        out_specs=pl.BlockSpec((2, 1, HW, C), lambda b: (0, b, 0, 0)),
        compiler_params=_cp("parallel"),
    )(imgs, imgs, flo)
    return (out[0].reshape(B, Hs, Ws, C), out[1].reshape(B, Hs, Ws, C))


def _nhwc(x):
    return jnp.transpose(x, (0, 2, 3, 1))


def _nchw(x):
    return jnp.transpose(x, (0, 3, 1, 2))


def kernel(enc1_w, enc1_b, enc2_w, enc2_b, enc3_w, enc3_b, enc4_w, enc4_b,
           dec4_w, dec4_b, qb3_w, qb3_b, dec3_w, dec3_b, dec2_w, dec2_b,
           dec1_w, dec1_b, upconv1_w, upconv1_b, hrconv_w, hrconv_b,
           convlast_w, convlast_b,
           qb2_ws0, qb2_ws1, qb2_ws2, qb2_ws3, qb2_b,
           qb1_ws0, qb1_ws1, qb1_ws2, qb1_ws3, qb1_b,
           x0, x1, t):
    B, C, H, W = x0.shape

    xcat, mean = _normalize(x0, x1)

    xin = _s2d(_pad1(_nhwc(xcat)))
    f1 = _conv([xin], [0], [_w_s2d(enc1_w, C, 32)], enc1_b, k=2, nb=2 * B,
               act="lrelu")
    f1p = _pad1(f1)
    f2 = _conv([_s2d(f1p)], [0], [_w_s2d(enc2_w, 32, 48)], enc2_b, k=2,
               nb=2 * B, act="lrelu")
    f2p = _pad1(f2)
    f3 = _conv([_s2d(f2p)], [0], [_w_s2d(enc3_w, 48, 72)], enc3_b, k=2,
               nb=2 * B, act="lrelu")
    f3p = _pad1(f3)
    f4 = _conv([f3p], [0], [enc4_w.reshape(3, 3, 72, 96)], enc4_b, k=3,
               nb=2 * B, act="lrelu")

    f4p = _pad1(f4)
    wd4 = dec4_w.reshape(3, 3, 192, 4)
    out4 = _conv([f4p, f4p], [0, B], [wd4[:, :, :96], wd4[:, :, 96:]], dec4_b,
                 k=3, nb=B, out_dtype=jnp.float32)
    f01 = _nchw(out4[..., 0:2])
    t4 = t.reshape(B, 1, 1, 1)
    ft0 = -(f01 * t4) * t4
    ft1 = -(f01 * (1.0 - t4)) * (1.0 - t4)

    warp0, warp1 = _bwarp2(f3, ft0, ft1)
    w0p = _pad1(warp0)
    w1p = _pad1(warp1)
    wq3 = qb3_w.reshape(3, 3, 144, 72)
    q3 = _conv([w0p, w1p], [0, 0], [wq3[:, :, :72], wq3[:, :, 72:]], qb3_b,
               k=3, nb=B)
    wd3 = dec3_w.reshape(3, 3, 216, 72)
    p3 = _conv([_pad1(q3), f3p, f3p], [0, 0, B],
               [wd3[:, :, :72], wd3[:, :, 72:144], wd3[:, :, 144:]], dec3_b,
               k=3, nb=B, act="lrelu")

    q2 = _conv_transpose(_pad1(p3),
                         [w.reshape(2, 2, 72, 48)
                          for w in (qb2_ws0, qb2_ws1, qb2_ws2, qb2_ws3)],
                         qb2_b)
    wd2 = dec2_w.reshape(3, 3, 144, 48)
    p2 = _conv([_pad1(q2), f2p, f2p], [0, 0, B],
               [wd2[:, :, :48], wd2[:, :, 48:96], wd2[:, :, 96:]], dec2_b,
               k=3, nb=B, act="lrelu")

    q1 = _conv_transpose(_pad1(p2),
                         [w.reshape(2, 2, 48, 32)
                          for w in (qb1_ws0, qb1_ws1, qb1_ws2, qb1_ws3)],
                         qb1_b)
    wd1 = dec1_w.reshape(3, 3, 96, 32)
    p1 = _conv([_pad1(q1), f1p, f1p], [0, 0, B],
               [wd1[:, :, :32], wd1[:, :, 32:64], wd1[:, :, 64:]], dec1_b,
               k=3, nb=B, act="lrelu")

    up = _conv([_pad1(p1)], [0], [upconv1_w.reshape(3, 3, 32, 128)],
               upconv1_b, k=3, nb=B, act="lrelu")
    ups = _pixel_shuffle(up, 2)
    hr = _conv([_pad1(ups)], [0], [hrconv_w.reshape(3, 3, 32, 32)], hrconv_b,
               k=3, nb=B, act="lrelu")
    mean_b = mean.reshape(B, C)[:, None, :]
    out = _conv([_pad1(hr)], [0], [convlast_w.reshape(3, 3, 32, 3)],
                convlast_b, k=3, nb=B, act="clamp", out_dtype=jnp.float32,
                mean=mean_b)
    return _nchw(out)
